# baseline (speedup 1.0000x reference)
import jax
import jax.numpy as jnp
from jax import lax
from jax.experimental import pallas as pl
from jax.experimental.pallas import tpu as pltpu
from jax.experimental.pallas import tpu_sc as plsc

B = 4096
L = 50
NF = 5
CONT = 8
LARGE_DIM = 64
SMALL_DIM = 16
LANES = 16

NC = 2
NS = 16
W = NC * NS
UPW = B // W
GR = UPW // LANES


def _sc_kernel(isp_hbm, icf_hbm, usp_hbm, ucf_hbm,
               ut0, ut1, ut2, ut3, ut4,
               it0, it1, it2, it3, it4,
               out_hbm,
               gidx, ugidx, icc, ucc,
               u0v, u1v, u2v, u3v, u4v,
               rows0, rows1, rows2, rows3, rows4,
               accb, outv, sem):
    wid = lax.axis_index("s") * NC + lax.axis_index("c")
    us = pl.ds(wid * UPW, UPW)

    for f in range(NF):
        pltpu.sync_copy(usp_hbm.at[f, us], ugidx.at[f])
    for c in range(CONT):
        pltpu.sync_copy(ucf_hbm.at[c, us], ucc.at[c])

    pltpu.async_copy(ut0.at[ugidx.at[0]], u0v, sem).wait()
    pltpu.async_copy(ut1.at[ugidx.at[1]], u1v, sem).wait()
    pltpu.async_copy(ut2.at[ugidx.at[2]], u2v, sem).wait()
    pltpu.async_copy(ut3.at[ugidx.at[3]], u3v, sem).wait()
    pltpu.async_copy(ut4.at[ugidx.at[4]], u4v, sem).wait()

    def cand_body(l, _):
        for f in range(NF):
            pltpu.sync_copy(isp_hbm.at[l, f, us], gidx.at[f])
        for c in range(CONT):
            pltpu.sync_copy(icf_hbm.at[l, c, us], icc.at[c])

        pltpu.async_copy(it0.at[gidx.at[0]], rows0, sem).wait()
        pltpu.async_copy(it1.at[gidx.at[1]], rows1, sem).wait()
        pltpu.async_copy(it2.at[gidx.at[2]], rows2, sem).wait()
        pltpu.async_copy(it3.at[gidx.at[3]], rows3, sem).wait()
        pltpu.async_copy(it4.at[gidx.at[4]], rows4, sem).wait()

        def slot_body(j, _):
            acc = u0v[j, pl.ds(0, 16)] * rows0[j, pl.ds(0, 16)]
            acc += u0v[j, pl.ds(16, 16)] * rows0[j, pl.ds(16, 16)]
            acc += u0v[j, pl.ds(32, 16)] * rows0[j, pl.ds(32, 16)]
            acc += u0v[j, pl.ds(48, 16)] * rows0[j, pl.ds(48, 16)]
            acc += u1v[j] * rows1[j]
            acc += u2v[j] * rows2[j]
            acc += u3v[j] * rows3[j]
            acc += u4v[j] * rows4[j]
            accb[pl.ds(j * LANES, LANES)] = acc
            return 0

        lax.fori_loop(0, UPW, slot_body, 0, unroll=2)

        def red_body(k, _):
            ks = pl.ds(k * LANES, LANES)
            tot = icc[0, ks] * ucc[0, ks]
            for c in range(1, CONT):
                tot = tot + icc[c, ks] * ucc[c, ks]
            rowbase = (k * LANES + lax.iota(jnp.int32, LANES)) * LANES
            for c in range(LANES):
                tot = tot + plsc.load_gather(accb, [rowbase + c])
            outv[l, ks] = tot
            return 0

        lax.fori_loop(0, GR, red_body, 0)
        return 0

    lax.fori_loop(0, L, cand_body, 0)

    pltpu.sync_copy(outv, out_hbm.at[:, us])


@jax.jit
def kernel(user_sparse, item_sparse, user_cont, item_cont,
           user_t0, user_t1, user_t2, user_t3, user_t4,
           item_t0, item_t1, item_t2, item_t3, item_t4):
    isp = item_sparse.transpose(1, 2, 0)
    icf = item_cont.transpose(1, 2, 0)
    usp = user_sparse.T
    ucf = user_cont.T

    mesh = plsc.VectorSubcoreMesh(core_axis_name="c", subcore_axis_name="s")
    run = pl.kernel(
        _sc_kernel,
        mesh=mesh,
        compiler_params=pltpu.CompilerParams(needs_layout_passes=False,
                                             use_tc_tiling_on_sc=False),
        out_type=jax.ShapeDtypeStruct((L, B), jnp.float32),
        scratch_types=[
            pltpu.VMEM((NF, UPW), jnp.int32),
            pltpu.VMEM((NF, UPW), jnp.int32),
            pltpu.VMEM((CONT, UPW), jnp.float32),
            pltpu.VMEM((CONT, UPW), jnp.float32),
            pltpu.VMEM((UPW, LARGE_DIM), jnp.float32),
            pltpu.VMEM((UPW, SMALL_DIM), jnp.float32),
            pltpu.VMEM((UPW, SMALL_DIM), jnp.float32),
            pltpu.VMEM((UPW, SMALL_DIM), jnp.float32),
            pltpu.VMEM((UPW, SMALL_DIM), jnp.float32),
            pltpu.VMEM((UPW, LARGE_DIM), jnp.float32),
            pltpu.VMEM((UPW, SMALL_DIM), jnp.float32),
            pltpu.VMEM((UPW, SMALL_DIM), jnp.float32),
            pltpu.VMEM((UPW, SMALL_DIM), jnp.float32),
            pltpu.VMEM((UPW, SMALL_DIM), jnp.float32),
            pltpu.VMEM((UPW * LANES,), jnp.float32),
            pltpu.VMEM((L, UPW), jnp.float32),
            pltpu.SemaphoreType.DMA,
        ],
    )
    out = run(isp, icf, usp, ucf,
              user_t0, user_t1, user_t2, user_t3, user_t4,
              item_t0, item_t1, item_t2, item_t3, item_t4)
    return out.T


# speedup vs baseline: 1.0392x; 1.0392x over previous
import jax
import jax.numpy as jnp
from jax import lax
from jax.experimental import pallas as pl
from jax.experimental.pallas import tpu as pltpu
from jax.experimental.pallas import tpu_sc as plsc

B = 4096
L = 50
NF = 5
CONT = 8
LARGE_DIM = 64
SMALL_DIM = 16
LANES = 16

NC = 2
NS = 16
W = NC * NS
UPW = B // W
SPW = UPW * L
CH = 128
NCH = SPW // CH
GR = CH // LANES


def _sc_kernel(iidx_hbm, icont_hbm, uidx_hbm, ucont_hbm,
               ut0, ut1, ut2, ut3, ut4,
               it0, it1, it2, it3, it4,
               out_hbm,
               idx_v, uidx_v, u0v, u1v, u2v, u3v, u4v, ucontv,
               rows0, rows1, rows2, rows3, rows4, icontv,
               accb, outv, sem):
    wid = lax.axis_index("s") * NC + lax.axis_index("c")

    pltpu.sync_copy(iidx_hbm.at[wid], idx_v)
    pltpu.sync_copy(uidx_hbm.at[wid], uidx_v)
    pltpu.sync_copy(ucont_hbm.at[wid], ucontv)

    pltpu.async_copy(ut0.at[uidx_v.at[0]], u0v, sem).wait()
    pltpu.async_copy(ut1.at[uidx_v.at[1]], u1v, sem).wait()
    pltpu.async_copy(ut2.at[uidx_v.at[2]], u2v, sem).wait()
    pltpu.async_copy(ut3.at[uidx_v.at[3]], u3v, sem).wait()
    pltpu.async_copy(ut4.at[uidx_v.at[4]], u4v, sem).wait()

    def chunk_body(g, _):
        pltpu.async_copy(it0.at[idx_v.at[0, g]], rows0, sem).wait()
        pltpu.async_copy(it1.at[idx_v.at[1, g]], rows1, sem).wait()
        pltpu.async_copy(it2.at[idx_v.at[2, g]], rows2, sem).wait()
        pltpu.async_copy(it3.at[idx_v.at[3, g]], rows3, sem).wait()
        pltpu.async_copy(it4.at[idx_v.at[4, g]], rows4, sem).wait()
        pltpu.sync_copy(icont_hbm.at[wid, g], icontv)

        def slot_body(j, _):
            lu = (g * CH + j) // L
            acc = ucontv[lu] * icontv[j]
            acc += u0v[lu, pl.ds(0, 16)] * rows0[j, pl.ds(0, 16)]
            acc += u0v[lu, pl.ds(16, 16)] * rows0[j, pl.ds(16, 16)]
            acc += u0v[lu, pl.ds(32, 16)] * rows0[j, pl.ds(32, 16)]
            acc += u0v[lu, pl.ds(48, 16)] * rows0[j, pl.ds(48, 16)]
            acc += u1v[lu] * rows1[j]
            acc += u2v[lu] * rows2[j]
            acc += u3v[lu] * rows3[j]
            acc += u4v[lu] * rows4[j]
            accb[pl.ds(j * LANES, LANES)] = acc
            return 0

        lax.fori_loop(0, CH, slot_body, 0, unroll=2)

        def red_body(k, _):
            rowbase = (k * LANES + lax.iota(jnp.int32, LANES)) * LANES
            tot = jnp.zeros((LANES,), jnp.float32)
            for c in range(LANES):
                tot = tot + plsc.load_gather(accb, [rowbase + c])
            outv[pl.ds(g * CH + k * LANES, LANES)] = tot
            return 0

        lax.fori_loop(0, GR, red_body, 0)
        return 0

    lax.fori_loop(0, NCH, chunk_body, 0)

    pltpu.sync_copy(outv, out_hbm.at[wid])


@jax.jit
def kernel(user_sparse, item_sparse, user_cont, item_cont,
           user_t0, user_t1, user_t2, user_t3, user_t4,
           item_t0, item_t1, item_t2, item_t3, item_t4):
    iidx = (item_sparse.reshape(B * L, NF)
            .reshape(W, NCH, CH, NF)
            .transpose(0, 3, 1, 2))
    icont = jnp.pad(item_cont.reshape(B * L, CONT),
                    ((0, 0), (0, LANES - CONT)))
    icont = icont.reshape(W, NCH, CH, LANES)
    uidx = user_sparse.reshape(W, UPW, NF).transpose(0, 2, 1)
    ucont = jnp.pad(user_cont, ((0, 0), (0, LANES - CONT)))
    ucont = ucont.reshape(W, UPW, LANES)

    mesh = plsc.VectorSubcoreMesh(core_axis_name="c", subcore_axis_name="s")
    run = pl.kernel(
        _sc_kernel,
        mesh=mesh,
        compiler_params=pltpu.CompilerParams(needs_layout_passes=False,
                                             use_tc_tiling_on_sc=False),
        out_type=jax.ShapeDtypeStruct((W, SPW), jnp.float32),
        scratch_types=[
            pltpu.VMEM((NF, NCH, CH), jnp.int32),
            pltpu.VMEM((NF, UPW), jnp.int32),
            pltpu.VMEM((UPW, LARGE_DIM), jnp.float32),
            pltpu.VMEM((UPW, SMALL_DIM), jnp.float32),
            pltpu.VMEM((UPW, SMALL_DIM), jnp.float32),
            pltpu.VMEM((UPW, SMALL_DIM), jnp.float32),
            pltpu.VMEM((UPW, SMALL_DIM), jnp.float32),
            pltpu.VMEM((UPW, LANES), jnp.float32),
            pltpu.VMEM((CH, LARGE_DIM), jnp.float32),
            pltpu.VMEM((CH, SMALL_DIM), jnp.float32),
            pltpu.VMEM((CH, SMALL_DIM), jnp.float32),
            pltpu.VMEM((CH, SMALL_DIM), jnp.float32),
            pltpu.VMEM((CH, SMALL_DIM), jnp.float32),
            pltpu.VMEM((CH, LANES), jnp.float32),
            pltpu.VMEM((CH * LANES,), jnp.float32),
            pltpu.VMEM((SPW,), jnp.float32),
            pltpu.SemaphoreType.DMA,
        ],
    )
    out = run(iidx, icont, uidx, ucont,
              user_t0, user_t1, user_t2, user_t3, user_t4,
              item_t0, item_t1, item_t2, item_t3, item_t4)
    return out.reshape(B, L)
